# Initial kernel scaffold; baseline (speedup 1.0000x reference)
import jax
import jax.numpy as jnp
from jax.experimental import pallas as pl


def kernel(h, edge_index):
    raise NotImplementedError("write your pallas kernel here")


# speedup vs baseline: 2.6317x; 2.6317x over previous
import functools

import jax
import jax.numpy as jnp
from jax import lax
from jax.experimental import pallas as pl
from jax.experimental.pallas import tpu as pltpu
from jax.experimental.pallas import tpu_sc as plsc

NC = 2
NS = 16
NW = NC * NS
L = 16
CHUNK = 128


@functools.lru_cache(maxsize=None)
def _make_kernel(E: int, D: int):
    assert E % CHUNK == 0 and D % L == 0
    n_chunks = E // CHUNK
    base_chunks = n_chunks // NW
    extra = n_chunks % NW

    mesh = plsc.VectorSubcoreMesh(core_axis_name="c", subcore_axis_name="s")

    @functools.partial(
        pl.kernel,
        out_type=jax.ShapeDtypeStruct((E,), jnp.float32),
        mesh=mesh,
        compiler_params=pltpu.CompilerParams(needs_layout_passes=False),
        scratch_types=[
            pltpu.VMEM((CHUNK,), jnp.int32),
            pltpu.VMEM((CHUNK,), jnp.int32),
            pltpu.VMEM((CHUNK, D), jnp.float32),
            pltpu.VMEM((CHUNK, D), jnp.float32),
            pltpu.VMEM((L * L,), jnp.float32),
            pltpu.VMEM((CHUNK,), jnp.float32),
            pltpu.SemaphoreType.DMA,
            pltpu.SemaphoreType.DMA,
        ],
    )
    def dot_scores(h_hbm, src_hbm, dst_hbm, out_hbm,
                   idx_s, idx_d, s_buf, d_buf, tmp, out_buf, sem_s, sem_d):
        wid = lax.axis_index("s") * NC + lax.axis_index("c")
        my_chunks = base_chunks + jnp.where(wid < extra, 1, 0)
        lane = lax.iota(jnp.int32, L)

        def chunk_body(i, carry):
            base = (wid + i * NW) * CHUNK
            pltpu.sync_copy(src_hbm.at[pl.ds(base, CHUNK)], idx_s)
            pltpu.sync_copy(dst_hbm.at[pl.ds(base, CHUNK)], idx_d)
            cp_s = pltpu.async_copy(h_hbm.at[idx_s], s_buf, sem_s)
            cp_d = pltpu.async_copy(h_hbm.at[idx_d], d_buf, sem_d)
            cp_s.wait()
            cp_d.wait()

            def group_body(g, gcarry):
                off = g * L
                for ee in range(L):
                    r = off + ee
                    acc = s_buf[r, pl.ds(0, L)] * d_buf[r, pl.ds(0, L)]
                    for k in range(1, D // L):
                        acc = acc + (s_buf[r, pl.ds(k * L, L)]
                                     * d_buf[r, pl.ds(k * L, L)])
                    tmp[pl.ds(ee * L, L)] = acc
                row_base = lane * L
                tot = plsc.load_gather(tmp, [row_base])
                for k in range(1, L):
                    tot = tot + plsc.load_gather(tmp, [row_base + k])
                out_buf[pl.ds(off, L)] = tot
                return gcarry

            lax.fori_loop(0, CHUNK // L, group_body, 0)
            pltpu.sync_copy(out_buf, out_hbm.at[pl.ds(base, CHUNK)])
            return carry

        lax.fori_loop(0, my_chunks, chunk_body, 0)

    return dot_scores


def kernel(h, edge_index):
    ei = edge_index.astype(jnp.int32)
    fn = _make_kernel(ei.shape[1], h.shape[1])
    return fn(h, ei[0], ei[1])


# speedup vs baseline: 5.6894x; 2.1619x over previous
import functools

import jax
import jax.numpy as jnp
from jax import lax
from jax.experimental import pallas as pl
from jax.experimental.pallas import tpu as pltpu
from jax.experimental.pallas import tpu_sc as plsc

NC = 2
NS = 16
NW = NC * NS
L = 16
CHUNK = 128


@functools.lru_cache(maxsize=None)
def _make_kernel(E: int, D: int):
    assert E % CHUNK == 0 and D % (2 * L) == 0
    n_chunks = E // CHUNK
    n_iters = -(-n_chunks // NW)
    n_pairs = -(-n_iters // 2)
    mesh = plsc.VectorSubcoreMesh(core_axis_name="c", subcore_axis_name="s")

    @functools.partial(
        pl.kernel,
        out_type=jax.ShapeDtypeStruct((E,), jnp.float32),
        mesh=mesh,
        compiler_params=pltpu.CompilerParams(
            needs_layout_passes=False,
            use_tc_tiling_on_sc=False,
        ),
        scratch_types=[
            pltpu.VMEM((CHUNK,), jnp.int32),
            pltpu.VMEM((CHUNK,), jnp.int32),
            pltpu.VMEM((CHUNK,), jnp.int32),
            pltpu.VMEM((CHUNK,), jnp.int32),
            pltpu.VMEM((CHUNK, D), jnp.bfloat16),
            pltpu.VMEM((CHUNK, D), jnp.bfloat16),
            pltpu.VMEM((CHUNK, D), jnp.bfloat16),
            pltpu.VMEM((CHUNK, D), jnp.bfloat16),
            pltpu.VMEM((L * L,), jnp.float32),
            pltpu.VMEM((CHUNK,), jnp.float32),
            pltpu.VMEM((CHUNK,), jnp.float32),
            pltpu.SemaphoreType.DMA,
            pltpu.SemaphoreType.DMA,
            pltpu.SemaphoreType.DMA,
            pltpu.SemaphoreType.DMA,
            pltpu.SemaphoreType.DMA,
            pltpu.SemaphoreType.DMA,
        ],
    )
    def dot_scores(h_hbm, src_hbm, dst_hbm, out_hbm,
                   ixs0, ixd0, ixs1, ixd1, s0, d0, s1, d1, tmp, ob0, ob1,
                   sem_i0, sem_i1, sem_g0, sem_g1, sem_o0, sem_o1):
        wid = lax.axis_index("s") * NC + lax.axis_index("c")
        lane = lax.iota(jnp.int32, L)
        last = n_chunks - 1

        def chunk_base(i):
            return jnp.minimum(wid + i * NW, last) * CHUNK

        def issue_idx(i, ixs, ixd, sem):
            base = chunk_base(i)
            pltpu.async_copy(src_hbm.at[pl.ds(base, CHUNK)], ixs, sem)
            pltpu.async_copy(dst_hbm.at[pl.ds(base, CHUNK)], ixd, sem)

        def wait_idx(ixs, ixd, sem):
            pltpu.make_async_copy(src_hbm.at[pl.ds(0, CHUNK)], ixs, sem).wait()
            pltpu.make_async_copy(dst_hbm.at[pl.ds(0, CHUNK)], ixd, sem).wait()

        def issue_gather(ixs, ixd, s_buf, d_buf, sem):
            pltpu.async_copy(h_hbm.at[ixs], s_buf, sem)
            pltpu.async_copy(h_hbm.at[ixd], d_buf, sem)

        def wait_gather(ixs, ixd, s_buf, d_buf, sem):
            pltpu.make_async_copy(h_hbm.at[ixs], s_buf, sem).wait()
            pltpu.make_async_copy(h_hbm.at[ixd], d_buf, sem).wait()

        def compute(s_buf, d_buf, out_buf):
            def group_body(g, gcarry):
                off = g * L
                for ee in range(L):
                    r = off + ee
                    prods = [
                        s_buf[r, pl.ds(k * 2 * L, 2 * L)]
                        * d_buf[r, pl.ds(k * 2 * L, 2 * L)]
                        for k in range(D // (2 * L))
                    ]
                    while len(prods) > 1:
                        prods = [prods[i] + prods[i + 1]
                                 for i in range(0, len(prods), 2)]
                    pa, pb = plsc.unpack(
                        prods[0], format=plsc.PackFormat.INTERLEAVED)
                    acc = pa + pb
                    tmp[pl.ds(ee * L, L)] = acc
                row_base = lane * L
                tot = plsc.load_gather(tmp, [row_base])
                for k in range(1, L):
                    tot = tot + plsc.load_gather(tmp, [row_base + k])
                out_buf[pl.ds(off, L)] = tot
                return gcarry

            lax.fori_loop(0, CHUNK // L, group_body, 0)

        def issue_out(i, out_buf, sem):
            base = chunk_base(i)
            pltpu.async_copy(out_buf, out_hbm.at[pl.ds(base, CHUNK)], sem)

        def wait_out(out_buf, sem):
            pltpu.make_async_copy(
                out_buf, out_hbm.at[pl.ds(0, CHUNK)], sem).wait()

        issue_idx(0, ixs0, ixd0, sem_i0)
        issue_idx(1, ixs1, ixd1, sem_i1)
        wait_idx(ixs0, ixd0, sem_i0)
        issue_gather(ixs0, ixd0, s0, d0, sem_g0)

        def pair_body(j, carry):
            i0 = 2 * j
            i1 = 2 * j + 1
            wait_gather(ixs0, ixd0, s0, d0, sem_g0)
            issue_idx(i0 + 2, ixs0, ixd0, sem_i0)
            wait_idx(ixs1, ixd1, sem_i1)
            issue_gather(ixs1, ixd1, s1, d1, sem_g1)

            @pl.when(j > 0)
            def _():
                wait_out(ob0, sem_o0)
            compute(s0, d0, ob0)
            issue_out(i0, ob0, sem_o0)

            wait_gather(ixs1, ixd1, s1, d1, sem_g1)
            wait_idx(ixs0, ixd0, sem_i0)
            issue_gather(ixs0, ixd0, s0, d0, sem_g0)
            issue_idx(i1 + 2, ixs1, ixd1, sem_i1)

            @pl.when(j > 0)
            def _():
                wait_out(ob1, sem_o1)
            compute(s1, d1, ob1)
            issue_out(i1, ob1, sem_o1)
            return carry

        lax.fori_loop(0, n_pairs, pair_body, 0)

        wait_gather(ixs0, ixd0, s0, d0, sem_g0)
        wait_idx(ixs1, ixd1, sem_i1)
        wait_out(ob0, sem_o0)
        wait_out(ob1, sem_o1)

    return dot_scores


def kernel(h, edge_index):
    ei = edge_index.astype(jnp.int32)
    fn = _make_kernel(ei.shape[1], h.shape[1])
    return fn(h.astype(jnp.bfloat16), ei[0], ei[1])


# speedup vs baseline: 6.8455x; 1.2032x over previous
import functools

import jax
import jax.numpy as jnp
from jax import lax
from jax.experimental import pallas as pl
from jax.experimental.pallas import tpu as pltpu
from jax.experimental.pallas import tpu_sc as plsc

NC = 2
NS = 16
NW = NC * NS
L = 16
CHUNK = 128


@functools.lru_cache(maxsize=None)
def _make_kernel(E: int, D: int):
    assert E % CHUNK == 0 and D % (2 * L) == 0
    n_chunks = E // CHUNK
    n_iters = -(-n_chunks // NW)
    n_pairs = -(-n_iters // 2)
    mesh = plsc.VectorSubcoreMesh(core_axis_name="c", subcore_axis_name="s")

    @functools.partial(
        pl.kernel,
        out_type=jax.ShapeDtypeStruct((E,), jnp.float32),
        mesh=mesh,
        compiler_params=pltpu.CompilerParams(
            needs_layout_passes=False,
            use_tc_tiling_on_sc=False,
        ),
        scratch_types=[
            pltpu.VMEM((CHUNK,), jnp.int32),
            pltpu.VMEM((CHUNK,), jnp.int32),
            pltpu.VMEM((CHUNK,), jnp.int32),
            pltpu.VMEM((CHUNK,), jnp.int32),
            pltpu.VMEM((CHUNK, D), jnp.bfloat16),
            pltpu.VMEM((CHUNK, D), jnp.bfloat16),
            pltpu.VMEM((CHUNK, D), jnp.bfloat16),
            pltpu.VMEM((CHUNK, D), jnp.bfloat16),
            pltpu.VMEM((CHUNK * L,), jnp.float32),
            pltpu.VMEM((CHUNK,), jnp.float32),
            pltpu.VMEM((CHUNK,), jnp.float32),
            pltpu.SemaphoreType.DMA,
            pltpu.SemaphoreType.DMA,
            pltpu.SemaphoreType.DMA,
            pltpu.SemaphoreType.DMA,
            pltpu.SemaphoreType.DMA,
            pltpu.SemaphoreType.DMA,
        ],
    )
    def dot_scores(h_hbm, src_hbm, dst_hbm, out_hbm,
                   ixs0, ixd0, ixs1, ixd1, s0, d0, s1, d1, tmp, ob0, ob1,
                   sem_i0, sem_i1, sem_g0, sem_g1, sem_o0, sem_o1):
        wid = lax.axis_index("s") * NC + lax.axis_index("c")
        lane = lax.iota(jnp.int32, L)
        last = n_chunks - 1

        def chunk_base(i):
            return jnp.minimum(wid + i * NW, last) * CHUNK

        def issue_idx(i, ixs, ixd, sem):
            base = chunk_base(i)
            pltpu.async_copy(src_hbm.at[pl.ds(base, CHUNK)], ixs, sem)
            pltpu.async_copy(dst_hbm.at[pl.ds(base, CHUNK)], ixd, sem)

        def wait_idx(ixs, ixd, sem):
            pltpu.make_async_copy(src_hbm.at[pl.ds(0, CHUNK)], ixs, sem).wait()
            pltpu.make_async_copy(dst_hbm.at[pl.ds(0, CHUNK)], ixd, sem).wait()

        def issue_gather(ixs, ixd, s_buf, d_buf, sem):
            pltpu.async_copy(h_hbm.at[ixs], s_buf, sem)
            pltpu.async_copy(h_hbm.at[ixd], d_buf, sem)

        def wait_gather(ixs, ixd, s_buf, d_buf, sem):
            pltpu.make_async_copy(h_hbm.at[ixs], s_buf, sem).wait()
            pltpu.make_async_copy(h_hbm.at[ixd], d_buf, sem).wait()

        def compute(s_buf, d_buf, out_buf):
            @plsc.parallel_loop(0, CHUNK, step=1, unroll=4)
            def edge_body(e):
                prods = [
                    s_buf[e, pl.ds(k * 2 * L, 2 * L)]
                    * d_buf[e, pl.ds(k * 2 * L, 2 * L)]
                    for k in range(D // (2 * L))
                ]
                while len(prods) > 1:
                    prods = [prods[i] + prods[i + 1]
                             for i in range(0, len(prods), 2)]
                pa, pb = plsc.unpack(
                    prods[0], format=plsc.PackFormat.INTERLEAVED)
                tmp[pl.ds(e * L, L)] = pa + pb

            @plsc.parallel_loop(0, CHUNK // L, step=1, unroll=2)
            def group_red(g):
                row_base = (g * L + lane) * L
                tot = plsc.load_gather(tmp, [row_base])
                for k in range(1, L):
                    tot = tot + plsc.load_gather(tmp, [row_base + k])
                out_buf[pl.ds(g * L, L)] = tot

        def issue_out(i, out_buf, sem):
            base = chunk_base(i)
            pltpu.async_copy(out_buf, out_hbm.at[pl.ds(base, CHUNK)], sem)

        def wait_out(out_buf, sem):
            pltpu.make_async_copy(
                out_buf, out_hbm.at[pl.ds(0, CHUNK)], sem).wait()

        issue_idx(0, ixs0, ixd0, sem_i0)
        issue_idx(1, ixs1, ixd1, sem_i1)
        wait_idx(ixs0, ixd0, sem_i0)
        issue_gather(ixs0, ixd0, s0, d0, sem_g0)

        def pair_body(j, carry):
            i0 = 2 * j
            i1 = 2 * j + 1
            wait_gather(ixs0, ixd0, s0, d0, sem_g0)
            issue_idx(i0 + 2, ixs0, ixd0, sem_i0)
            wait_idx(ixs1, ixd1, sem_i1)
            issue_gather(ixs1, ixd1, s1, d1, sem_g1)

            @pl.when(j > 0)
            def _():
                wait_out(ob0, sem_o0)
            compute(s0, d0, ob0)
            issue_out(i0, ob0, sem_o0)

            wait_gather(ixs1, ixd1, s1, d1, sem_g1)
            wait_idx(ixs0, ixd0, sem_i0)
            issue_gather(ixs0, ixd0, s0, d0, sem_g0)
            issue_idx(i1 + 2, ixs1, ixd1, sem_i1)

            @pl.when(j > 0)
            def _():
                wait_out(ob1, sem_o1)
            compute(s1, d1, ob1)
            issue_out(i1, ob1, sem_o1)
            return carry

        lax.fori_loop(0, n_pairs, pair_body, 0)

        wait_gather(ixs0, ixd0, s0, d0, sem_g0)
        wait_idx(ixs1, ixd1, sem_i1)
        wait_out(ob0, sem_o0)
        wait_out(ob1, sem_o1)

    return dot_scores


def kernel(h, edge_index):
    ei = edge_index.astype(jnp.int32)
    fn = _make_kernel(ei.shape[1], h.shape[1])
    return fn(h.astype(jnp.bfloat16), ei[0], ei[1])


# speedup vs baseline: 7.8614x; 1.1484x over previous
import functools

import jax
import jax.numpy as jnp
from jax import lax
from jax.experimental import pallas as pl
from jax.experimental.pallas import tpu as pltpu
from jax.experimental.pallas import tpu_sc as plsc

NC = 2
NS = 16
NW = NC * NS
L = 16
CHUNK = 64


@functools.lru_cache(maxsize=None)
def _make_kernel(E: int, D: int, N: int):
    assert E % CHUNK == 0 and D % (2 * L) == 0
    n_chunks = E // CHUNK
    n_iters = -(-n_chunks // NW)
    n_pairs = -(-n_iters // 2)
    mesh = plsc.VectorSubcoreMesh(core_axis_name="c", subcore_axis_name="s")

    @functools.partial(
        pl.kernel,
        out_type=jax.ShapeDtypeStruct((E,), jnp.float32),
        mesh=mesh,
        compiler_params=pltpu.CompilerParams(
            needs_layout_passes=False,
            use_tc_tiling_on_sc=False,
        ),
        scratch_types=[
            pltpu.VMEM((CHUNK,), jnp.int32),
            pltpu.VMEM((CHUNK,), jnp.int32),
            pltpu.VMEM((CHUNK,), jnp.int32),
            pltpu.VMEM((CHUNK,), jnp.int32),
            pltpu.VMEM((CHUNK, D), jnp.bfloat16),
            pltpu.VMEM((CHUNK, D), jnp.bfloat16),
            pltpu.VMEM((CHUNK, D), jnp.bfloat16),
            pltpu.VMEM((CHUNK, D), jnp.bfloat16),
            pltpu.VMEM((CHUNK * L,), jnp.float32),
            pltpu.VMEM((CHUNK,), jnp.float32),
            pltpu.VMEM((CHUNK,), jnp.float32),
            pltpu.VMEM_SHARED((N, D), jnp.bfloat16),
            pltpu.SemaphoreType.DMA,
            pltpu.SemaphoreType.DMA,
            pltpu.SemaphoreType.DMA,
            pltpu.SemaphoreType.DMA,
            pltpu.SemaphoreType.DMA,
            pltpu.SemaphoreType.DMA,
            pltpu.SemaphoreType.DMA,
        ],
    )
    def dot_scores(h_hbm, src_hbm, dst_hbm, out_hbm,
                   ixs0, ixd0, ixs1, ixd1, s0, d0, s1, d1, tmp, ob0, ob1,
                   table, sem_t,
                   sem_i0, sem_i1, sem_g0, sem_g1, sem_o0, sem_o1):
        wid = lax.axis_index("s") * NC + lax.axis_index("c")
        lane = lax.iota(jnp.int32, L)
        last = n_chunks - 1

        @pl.when(lax.axis_index("s") == 0)
        def _():
            pltpu.async_copy(h_hbm, table, sem_t)
            pltpu.make_async_copy(h_hbm, table, sem_t).wait()
        plsc.subcore_barrier()

        def chunk_base(i):
            return jnp.minimum(wid + i * NW, last) * CHUNK

        def issue_idx(i, ixs, ixd, sem):
            base = chunk_base(i)
            pltpu.async_copy(src_hbm.at[pl.ds(base, CHUNK)], ixs, sem)
            pltpu.async_copy(dst_hbm.at[pl.ds(base, CHUNK)], ixd, sem)

        def wait_idx(ixs, ixd, sem):
            pltpu.make_async_copy(src_hbm.at[pl.ds(0, CHUNK)], ixs, sem).wait()
            pltpu.make_async_copy(dst_hbm.at[pl.ds(0, CHUNK)], ixd, sem).wait()

        def issue_gather(ixs, ixd, s_buf, d_buf, sem):
            pltpu.async_copy(table.at[ixs], s_buf, sem)
            pltpu.async_copy(table.at[ixd], d_buf, sem)

        def wait_gather(ixs, ixd, s_buf, d_buf, sem):
            pltpu.make_async_copy(table.at[ixs], s_buf, sem).wait()
            pltpu.make_async_copy(table.at[ixd], d_buf, sem).wait()

        def compute(s_buf, d_buf, out_buf):
            @plsc.parallel_loop(0, CHUNK, step=1, unroll=4)
            def edge_body(e):
                prods = [
                    s_buf[e, pl.ds(k * 2 * L, 2 * L)]
                    * d_buf[e, pl.ds(k * 2 * L, 2 * L)]
                    for k in range(D // (2 * L))
                ]
                while len(prods) > 1:
                    prods = [prods[i] + prods[i + 1]
                             for i in range(0, len(prods), 2)]
                pa, pb = plsc.unpack(
                    prods[0], format=plsc.PackFormat.INTERLEAVED)
                tmp[pl.ds(e * L, L)] = pa + pb

            @plsc.parallel_loop(0, CHUNK // L, step=1, unroll=2)
            def group_red(g):
                row_base = (g * L + lane) * L
                tot = plsc.load_gather(tmp, [row_base])
                for k in range(1, L):
                    tot = tot + plsc.load_gather(tmp, [row_base + k])
                out_buf[pl.ds(g * L, L)] = tot

        def issue_out(i, out_buf, sem):
            base = chunk_base(i)
            pltpu.async_copy(out_buf, out_hbm.at[pl.ds(base, CHUNK)], sem)

        def wait_out(out_buf, sem):
            pltpu.make_async_copy(
                out_buf, out_hbm.at[pl.ds(0, CHUNK)], sem).wait()

        issue_idx(0, ixs0, ixd0, sem_i0)
        issue_idx(1, ixs1, ixd1, sem_i1)
        wait_idx(ixs0, ixd0, sem_i0)
        issue_gather(ixs0, ixd0, s0, d0, sem_g0)

        def pair_body(j, carry):
            i0 = 2 * j
            i1 = 2 * j + 1
            wait_gather(ixs0, ixd0, s0, d0, sem_g0)
            issue_idx(i0 + 2, ixs0, ixd0, sem_i0)
            wait_idx(ixs1, ixd1, sem_i1)
            issue_gather(ixs1, ixd1, s1, d1, sem_g1)

            @pl.when(j > 0)
            def _():
                wait_out(ob0, sem_o0)
            compute(s0, d0, ob0)
            issue_out(i0, ob0, sem_o0)

            wait_gather(ixs1, ixd1, s1, d1, sem_g1)
            wait_idx(ixs0, ixd0, sem_i0)
            issue_gather(ixs0, ixd0, s0, d0, sem_g0)
            issue_idx(i1 + 2, ixs1, ixd1, sem_i1)

            @pl.when(j > 0)
            def _():
                wait_out(ob1, sem_o1)
            compute(s1, d1, ob1)
            issue_out(i1, ob1, sem_o1)
            return carry

        lax.fori_loop(0, n_pairs, pair_body, 0)

        wait_gather(ixs0, ixd0, s0, d0, sem_g0)
        wait_idx(ixs1, ixd1, sem_i1)
        wait_out(ob0, sem_o0)
        wait_out(ob1, sem_o1)

    return dot_scores


def kernel(h, edge_index):
    ei = edge_index.astype(jnp.int32)
    fn = _make_kernel(ei.shape[1], h.shape[1], h.shape[0])
    return fn(h.astype(jnp.bfloat16), ei[0], ei[1])


# speedup vs baseline: 7.8970x; 1.0045x over previous
import functools

import jax
import jax.numpy as jnp
from jax import lax
from jax.experimental import pallas as pl
from jax.experimental.pallas import tpu as pltpu
from jax.experimental.pallas import tpu_sc as plsc

NC = 2
NS = 16
NW = NC * NS
L = 16
CHUNK = 64


@functools.lru_cache(maxsize=None)
def _make_kernel(E: int, D: int, N: int):
    assert E % CHUNK == 0 and D % (2 * L) == 0
    n_chunks = E // CHUNK
    n_iters = -(-n_chunks // NW)
    n_pairs = -(-n_iters // 2)
    mesh = plsc.VectorSubcoreMesh(core_axis_name="c", subcore_axis_name="s")

    @functools.partial(
        pl.kernel,
        out_type=jax.ShapeDtypeStruct((E,), jnp.float32),
        mesh=mesh,
        compiler_params=pltpu.CompilerParams(
            needs_layout_passes=False,
            use_tc_tiling_on_sc=False,
        ),
        scratch_types=[
            pltpu.VMEM((2 * CHUNK,), jnp.int32),
            pltpu.VMEM((2 * CHUNK,), jnp.int32),
            pltpu.VMEM((2 * CHUNK, D), jnp.bfloat16),
            pltpu.VMEM((2 * CHUNK, D), jnp.bfloat16),
            pltpu.VMEM((CHUNK * L,), jnp.float32),
            pltpu.VMEM((CHUNK,), jnp.float32),
            pltpu.VMEM((CHUNK,), jnp.float32),
            pltpu.VMEM_SHARED((N, D), jnp.bfloat16),
            pltpu.SemaphoreType.DMA,
            pltpu.SemaphoreType.DMA,
            pltpu.SemaphoreType.DMA,
            pltpu.SemaphoreType.DMA,
            pltpu.SemaphoreType.DMA,
            pltpu.SemaphoreType.DMA,
            pltpu.SemaphoreType.DMA,
        ],
    )
    def dot_scores(h_hbm, src_hbm, dst_hbm, out_hbm,
                   ix0, ix1, r0, r1, tmp, ob0, ob1,
                   table, sem_t,
                   sem_i0, sem_i1, sem_g0, sem_g1, sem_o0, sem_o1):
        wid = lax.axis_index("s") * NC + lax.axis_index("c")
        lane = lax.iota(jnp.int32, L)
        last = n_chunks - 1

        @pl.when(lax.axis_index("s") == 0)
        def _():
            pltpu.async_copy(h_hbm, table, sem_t)
            pltpu.make_async_copy(h_hbm, table, sem_t).wait()
        plsc.subcore_barrier()

        def chunk_base(i):
            return jnp.minimum(wid + i * NW, last) * CHUNK

        def issue_idx(i, ix, sem):
            base = chunk_base(i)
            pltpu.async_copy(
                src_hbm.at[pl.ds(base, CHUNK)], ix.at[pl.ds(0, CHUNK)], sem)
            pltpu.async_copy(
                dst_hbm.at[pl.ds(base, CHUNK)], ix.at[pl.ds(CHUNK, CHUNK)],
                sem)

        def wait_idx(ix, sem):
            pltpu.make_async_copy(
                src_hbm.at[pl.ds(0, CHUNK)], ix.at[pl.ds(0, CHUNK)],
                sem).wait()
            pltpu.make_async_copy(
                dst_hbm.at[pl.ds(0, CHUNK)], ix.at[pl.ds(CHUNK, CHUNK)],
                sem).wait()

        def issue_gather(ix, buf, sem):
            pltpu.async_copy(table.at[ix], buf, sem)

        def wait_gather(ix, buf, sem):
            pltpu.make_async_copy(table.at[ix], buf, sem).wait()

        def compute(buf, out_buf):
            @plsc.parallel_loop(0, CHUNK, step=1, unroll=4)
            def edge_body(e):
                prods = [
                    buf[e, pl.ds(k * 2 * L, 2 * L)]
                    * buf[CHUNK + e, pl.ds(k * 2 * L, 2 * L)]
                    for k in range(D // (2 * L))
                ]
                while len(prods) > 1:
                    prods = [prods[i] + prods[i + 1]
                             for i in range(0, len(prods), 2)]
                pa, pb = plsc.unpack(
                    prods[0], format=plsc.PackFormat.INTERLEAVED)
                tmp[pl.ds(e * L, L)] = pa + pb

            @plsc.parallel_loop(0, CHUNK // L, step=1, unroll=2)
            def group_red(g):
                row_base = (g * L + lane) * L
                tot = plsc.load_gather(tmp, [row_base])
                for k in range(1, L):
                    tot = tot + plsc.load_gather(tmp, [row_base + k])
                out_buf[pl.ds(g * L, L)] = tot

        def issue_out(i, out_buf, sem):
            base = chunk_base(i)
            pltpu.async_copy(out_buf, out_hbm.at[pl.ds(base, CHUNK)], sem)

        def wait_out(out_buf, sem):
            pltpu.make_async_copy(
                out_buf, out_hbm.at[pl.ds(0, CHUNK)], sem).wait()

        issue_idx(0, ix0, sem_i0)
        issue_idx(1, ix1, sem_i1)
        wait_idx(ix0, sem_i0)
        issue_gather(ix0, r0, sem_g0)

        def pair_body(j, carry):
            i0 = 2 * j
            i1 = 2 * j + 1
            wait_gather(ix0, r0, sem_g0)
            issue_idx(i0 + 2, ix0, sem_i0)
            wait_idx(ix1, sem_i1)
            issue_gather(ix1, r1, sem_g1)

            @pl.when(j > 0)
            def _():
                wait_out(ob0, sem_o0)
            compute(r0, ob0)
            issue_out(i0, ob0, sem_o0)

            wait_gather(ix1, r1, sem_g1)
            wait_idx(ix0, sem_i0)
            issue_gather(ix0, r0, sem_g0)
            issue_idx(i1 + 2, ix1, sem_i1)

            @pl.when(j > 0)
            def _():
                wait_out(ob1, sem_o1)
            compute(r1, ob1)
            issue_out(i1, ob1, sem_o1)
            return carry

        lax.fori_loop(0, n_pairs, pair_body, 0)

        wait_gather(ix0, r0, sem_g0)
        wait_idx(ix1, sem_i1)
        wait_out(ob0, sem_o0)
        wait_out(ob1, sem_o1)

    return dot_scores


def kernel(h, edge_index):
    ei = edge_index.astype(jnp.int32)
    fn = _make_kernel(ei.shape[1], h.shape[1], h.shape[0])
    return fn(h.astype(jnp.bfloat16), ei[0], ei[1])
